# Initial kernel scaffold; baseline (speedup 1.0000x reference)
import jax
import jax.numpy as jnp
from jax.experimental import pallas as pl


def kernel(x, edge_index, W0_lin, W0_film, b0_film, W0_skip, W0_fskip, W1_lin, W1_film, b1_film, W1_skip, W1_fskip, W2_lin, W2_film, b2_film, W2_skip, W2_fskip, bn0_g, bn0_b, bn1_g, bn1_b):
    raise NotImplementedError("write your pallas kernel here")


# speedup vs baseline: 3.6321x; 3.6321x over previous
import functools

import jax
import jax.numpy as jnp
from jax import lax
from jax.experimental import pallas as pl
from jax.experimental.pallas import tpu as pltpu
from jax.experimental.pallas import tpu_sc as plsc

_N = 10000
_E = 320000
_D = 128
_EPS = 1e-5

_NC = 2
_NS = 16
_NW = _NC * _NS
_EPW = _E // _NW
_CH = 80
_NCHUNK = _EPW // _CH
_NP = 10240
_RPT = _NP // _NS

_NB = 10
_BLK = _N // _NB


def _sc_edge_film(xl, gb, src3, dst3, zeros, zeros16, ch, with_count):
    nchunk = _EPW // ch
    src3 = src3.reshape(_NW, nchunk, ch)
    dst3 = dst3.reshape(_NW, nchunk, ch)
    mesh = plsc.VectorSubcoreMesh(core_axis_name="c", subcore_axis_name="s")
    outs = [jax.ShapeDtypeStruct((_NC, _NP, _D), jnp.float32)]
    scratch = [
        pltpu.VMEM((ch,), jnp.int32),
        pltpu.VMEM((ch,), jnp.int32),
        pltpu.VMEM((ch, _D), jnp.float32),
        pltpu.VMEM((ch, 2 * _D), jnp.float32),
        pltpu.VMEM((ch, _D), jnp.float32),
        pltpu.VMEM_SHARED((_NP, _D), jnp.float32),
    ]
    args = [xl, gb, src3, dst3, zeros]
    if with_count:
        outs.append(jax.ShapeDtypeStruct((_NC, _NP, 16), jnp.float32))
        scratch += [pltpu.VMEM((ch, 16), jnp.float32),
                    pltpu.VMEM_SHARED((_NP, 16), jnp.float32)]
        args.append(zeros16)

    @functools.partial(
        pl.kernel,
        out_type=tuple(outs) if with_count else outs[0],
        mesh=mesh,
        compiler_params=pltpu.CompilerParams(use_tc_tiling_on_sc=False),
        scratch_types=scratch,
    )
    def k(*refs):
        if with_count:
            (xl_hbm, gb_hbm, src_hbm, dst_hbm, zeros_hbm, zeros16_hbm,
             out_hbm, cnt_hbm, src_v, dst_v, xs_v, gb_v, msg_v, acc_sh,
             ones_v, cnt_sh) = refs
        else:
            (xl_hbm, gb_hbm, src_hbm, dst_hbm, zeros_hbm,
             out_hbm, src_v, dst_v, xs_v, gb_v, msg_v, acc_sh) = refs
        c = lax.axis_index("c")
        s = lax.axis_index("s")
        wid = c * _NS + s
        pltpu.sync_copy(zeros_hbm, acc_sh.at[pl.ds(s * _RPT, _RPT)])
        one0 = jnp.maximum(1.0 - lax.iota(jnp.int32, 16).astype(jnp.float32),
                           0.0)
        if with_count:
            pltpu.sync_copy(zeros16_hbm, cnt_sh.at[pl.ds(s * _RPT, _RPT)])

            def fill(e, carry):
                ones_v[e, pl.ds(0, 16)] = one0
                return carry

            lax.fori_loop(0, ch, fill, 0)
        plsc.subcore_barrier()

        def chunk(j, carry):
            pltpu.sync_copy(src_hbm.at[wid, j], src_v)
            pltpu.sync_copy(dst_hbm.at[wid, j], dst_v)
            pltpu.sync_copy(xl_hbm.at[src_v], xs_v)
            pltpu.sync_copy(gb_hbm.at[dst_v], gb_v)

            def edge(e, carry2):
                for cc in range(8):
                    g = gb_v[e, pl.ds(cc * 16, 16)]
                    b = gb_v[e, pl.ds(_D + cc * 16, 16)]
                    xv = xs_v[e, pl.ds(cc * 16, 16)]
                    msg_v[e, pl.ds(cc * 16, 16)] = jnp.maximum(g * xv + b, 0.0)
                return carry2

            lax.fori_loop(0, ch, edge, 0)
            pltpu.sync_copy(msg_v, acc_sh.at[dst_v], add=True)
            if with_count:
                pltpu.sync_copy(ones_v, cnt_sh.at[dst_v], add=True)
            return carry

        lax.fori_loop(0, nchunk, chunk, 0)
        plsc.subcore_barrier()
        pltpu.sync_copy(acc_sh.at[pl.ds(s * _RPT, _RPT)],
                        out_hbm.at[c, pl.ds(s * _RPT, _RPT)])
        if with_count:
            pltpu.sync_copy(cnt_sh.at[pl.ds(s * _RPT, _RPT)],
                            cnt_hbm.at[c, pl.ds(s * _RPT, _RPT)])

    return k(*args)


def _sc_edge_sum(xl, src3, dst3, zeros):
    src3 = src3.reshape(_NW, _NCHUNK, _CH)
    dst3 = dst3.reshape(_NW, _NCHUNK, _CH)
    mesh = plsc.VectorSubcoreMesh(core_axis_name="c", subcore_axis_name="s")

    @functools.partial(
        pl.kernel,
        out_type=jax.ShapeDtypeStruct((_NC, _NP, _D), jnp.float32),
        mesh=mesh,
        compiler_params=pltpu.CompilerParams(use_tc_tiling_on_sc=False),
        scratch_types=[
            pltpu.VMEM((_NCHUNK, _CH), jnp.int32),
            pltpu.VMEM((_NCHUNK, _CH), jnp.int32),
            pltpu.VMEM((_CH, _D), jnp.float32),
            pltpu.VMEM_SHARED((_NP, _D), jnp.float32),
        ],
    )
    def k(xl_hbm, src_hbm, dst_hbm, zeros_hbm, out_hbm,
          src_v, dst_v, xs_v, acc_sh):
        c = lax.axis_index("c")
        s = lax.axis_index("s")
        wid = c * _NS + s
        pltpu.sync_copy(zeros_hbm, acc_sh.at[pl.ds(s * _RPT, _RPT)])
        pltpu.sync_copy(src_hbm.at[wid], src_v)
        pltpu.sync_copy(dst_hbm.at[wid], dst_v)
        plsc.subcore_barrier()

        def chunk(j, carry):
            pltpu.sync_copy(xl_hbm.at[src_v.at[j]], xs_v)
            pltpu.sync_copy(xs_v, acc_sh.at[dst_v.at[j]], add=True)
            return carry

        lax.fori_loop(0, _NCHUNK, chunk, 0)
        plsc.subcore_barrier()
        pltpu.sync_copy(acc_sh.at[pl.ds(s * _RPT, _RPT)],
                        out_hbm.at[c, pl.ds(s * _RPT, _RPT)])

    return k(xl, src3, dst3, zeros)


def _dense(h, Wcat, bfilm, stats, bn_g, bn_b, act):
    with_bn = stats is not None
    args = [h, Wcat, bfilm.reshape(1, 2 * _D)]
    in_specs = [
        pl.BlockSpec((_BLK, _D), lambda i: (i, 0)),
        pl.BlockSpec((_D, 6 * _D), lambda i: (0, 0)),
        pl.BlockSpec((1, 2 * _D), lambda i: (0, 0)),
    ]
    if with_bn:
        args += [stats, bn_g.reshape(1, _D), bn_b.reshape(1, _D)]
        in_specs += [
            pl.BlockSpec((8, _D), lambda i: (0, 0)),
            pl.BlockSpec((1, _D), lambda i: (0, 0)),
            pl.BlockSpec((1, _D), lambda i: (0, 0)),
        ]

    def body(*refs):
        if with_bn:
            h_ref, w_ref, bf_ref, st_ref, g_ref, b_ref, xl_ref, gb_ref, sk_ref = refs
            m = st_ref[0, :] * (1.0 / _N)
            ex2 = st_ref[1, :] * (1.0 / _N)
            rstd = lax.rsqrt(jnp.maximum(ex2 - m * m, 0.0) + _EPS)
            sc = rstd * g_ref[0, :]
            hb = (h_ref[...] - m[None, :]) * sc[None, :] + b_ref[0, :][None, :]
        else:
            h_ref, w_ref, bf_ref, xl_ref, gb_ref, sk_ref = refs
            hb = h_ref[...]
        z = jnp.dot(hb, w_ref[...], preferred_element_type=jnp.float32)
        xl_ref[...] = z[:, :_D]
        beta = z[:, _D:2 * _D] + bf_ref[0, :_D][None, :]
        gamma = z[:, 2 * _D:3 * _D] + bf_ref[0, _D:][None, :]
        gb_ref[...] = jnp.concatenate([gamma, beta], axis=1)
        sk = z[:, 5 * _D:6 * _D] * z[:, 3 * _D:4 * _D] + z[:, 4 * _D:5 * _D]
        if act:
            sk = jnp.maximum(sk, 0.0)
        sk_ref[...] = sk

    return pl.pallas_call(
        body,
        grid=(_NB,),
        in_specs=in_specs,
        out_specs=[
            pl.BlockSpec((_BLK, _D), lambda i: (i, 0)),
            pl.BlockSpec((_BLK, 2 * _D), lambda i: (i, 0)),
            pl.BlockSpec((_BLK, _D), lambda i: (i, 0)),
        ],
        out_shape=[
            jax.ShapeDtypeStruct((_N, _D), jnp.float32),
            jax.ShapeDtypeStruct((_N, 2 * _D), jnp.float32),
            jax.ShapeDtypeStruct((_N, _D), jnp.float32),
        ],
        compiler_params=pltpu.CompilerParams(
            dimension_semantics=("arbitrary",)),
    )(*args)


def _combine(skip, sc_acc, cnt_sc, cnt2):
    first = cnt2 is None
    args = [skip, sc_acc, cnt_sc] if first else [skip, sc_acc, cnt2]
    in_specs = [
        pl.BlockSpec((_BLK, _D), lambda i: (i, 0)),
        pl.BlockSpec((2, _BLK, _D), lambda i: (0, i, 0)),
    ]
    if first:
        in_specs.append(pl.BlockSpec((2, _BLK, 16), lambda i: (0, i, 0)))
    else:
        in_specs.append(pl.BlockSpec((_BLK, _D), lambda i: (i, 0)))

    def body(*refs):
        if first:
            sk_ref, sc_ref, csc_ref, y_ref, cnt_ref, st_ref, acc = refs
        else:
            sk_ref, sc_ref, cin_ref, y_ref, st_ref, acc = refs
        i = pl.program_id(0)
        msum = sc_ref[0, :, :] + sc_ref[1, :, :]
        if first:
            cnt = csc_ref[0, :, 0] + csc_ref[1, :, 0]
            cnt2b = jnp.broadcast_to(cnt[:, None], (_BLK, _D))
            cnt_ref[...] = cnt2b
        else:
            cnt2b = cin_ref[...]
        y = sk_ref[...] + msum / jnp.maximum(cnt2b, 1.0)
        y_ref[...] = y

        @pl.when(i == 0)
        def _():
            acc[...] = jnp.zeros_like(acc)

        acc[0:1, :] += jnp.sum(y, axis=0)[None, :]
        acc[1:2, :] += jnp.sum(y * y, axis=0)[None, :]

        @pl.when(i == _NB - 1)
        def _():
            st_ref[...] = acc[...]

    out_specs = [pl.BlockSpec((_BLK, _D), lambda i: (i, 0))]
    out_shape = [jax.ShapeDtypeStruct((_N, _D), jnp.float32)]
    if first:
        out_specs.append(pl.BlockSpec((_BLK, _D), lambda i: (i, 0)))
        out_shape.append(jax.ShapeDtypeStruct((_N, _D), jnp.float32))
    out_specs.append(pl.BlockSpec((8, _D), lambda i: (0, 0)))
    out_shape.append(jax.ShapeDtypeStruct((8, _D), jnp.float32))

    return pl.pallas_call(
        body,
        grid=(_NB,),
        in_specs=in_specs,
        out_specs=out_specs,
        out_shape=out_shape,
        scratch_shapes=[pltpu.VMEM((8, _D), jnp.float32)],
        compiler_params=pltpu.CompilerParams(
            dimension_semantics=("arbitrary",)),
    )(*args)


def _final(skip, sc_acc, gb, cnt2):
    def body(sk_ref, sc_ref, gb_ref, cnt_ref, o_ref):
        S = sc_ref[0, :, :] + sc_ref[1, :, :]
        cnt = cnt_ref[...]
        gamma = gb_ref[:, :_D]
        beta = gb_ref[:, _D:]
        o_ref[...] = (sk_ref[...] + gamma * S / jnp.maximum(cnt, 1.0)
                      + beta * (cnt > 0.0).astype(jnp.float32))

    return pl.pallas_call(
        body,
        grid=(_NB,),
        in_specs=[
            pl.BlockSpec((_BLK, _D), lambda i: (i, 0)),
            pl.BlockSpec((2, _BLK, _D), lambda i: (0, i, 0)),
            pl.BlockSpec((_BLK, 2 * _D), lambda i: (i, 0)),
            pl.BlockSpec((_BLK, _D), lambda i: (i, 0)),
        ],
        out_specs=pl.BlockSpec((_BLK, _D), lambda i: (i, 0)),
        out_shape=jax.ShapeDtypeStruct((_N, _D), jnp.float32),
        compiler_params=pltpu.CompilerParams(
            dimension_semantics=("arbitrary",)),
    )(skip, sc_acc, gb, cnt2)


def kernel(x, edge_index, W0_lin, W0_film, b0_film, W0_skip, W0_fskip,
           W1_lin, W1_film, b1_film, W1_skip, W1_fskip,
           W2_lin, W2_film, b2_film, W2_skip, W2_fskip,
           bn0_g, bn0_b, bn1_g, bn1_b):
    src_ids = edge_index[0]
    dst_ids = edge_index[1]
    z128 = jnp.zeros((_RPT, _D), jnp.float32)
    z16 = jnp.zeros((_RPT, 16), jnp.float32)
    Wcat0 = jnp.concatenate([W0_lin, W0_film, W0_skip, W0_fskip], axis=1)
    Wcat1 = jnp.concatenate([W1_lin, W1_film, W1_skip, W1_fskip], axis=1)
    Wcat2 = jnp.concatenate([W2_lin, W2_film, W2_skip, W2_fskip], axis=1)

    xl0, gb0, skip0 = _dense(x, Wcat0, b0_film, None, None, None, act=True)
    sc0, cnt_sc = _sc_edge_film(xl0, gb0, src_ids, dst_ids, z128, z16,
                                ch=40, with_count=True)
    y0, cnt2, stats0 = _combine(skip0, sc0, cnt_sc, None)

    xl1, gb1, skip1 = _dense(y0, Wcat1, b1_film, stats0, bn0_g, bn0_b,
                             act=True)
    sc1 = _sc_edge_film(xl1, gb1, src_ids, dst_ids, z128, None,
                        ch=_CH, with_count=False)
    y1, stats1 = _combine(skip1, sc1, None, cnt2)

    xl2, gb2, skip2 = _dense(y1, Wcat2, b2_film, stats1, bn1_g, bn1_b,
                             act=False)
    sc2 = _sc_edge_sum(xl2, src_ids, dst_ids, z128)
    return _final(skip2, sc2, gb2, cnt2)


# speedup vs baseline: 5.0128x; 1.3802x over previous
import functools

import jax
import jax.numpy as jnp
from jax import lax
from jax.experimental import pallas as pl
from jax.experimental.pallas import tpu as pltpu
from jax.experimental.pallas import tpu_sc as plsc

_N = 10000
_E = 320000
_D = 128
_EPS = 1e-5

_NC = 2
_NS = 16
_NW = _NC * _NS
_EPW = _E // _NW
_CH = 80
_NCHUNK = _EPW // _CH
_NP = 10240
_RPT = _NP // _NS

_NB = 10
_BLK = _N // _NB


def _sc_edge_film(xl, gb, src3, dst3, zeros, zeros16, ch, with_count):
    nchunk = _EPW // ch
    src3 = src3.reshape(_NW, nchunk, ch)
    dst3 = dst3.reshape(_NW, nchunk, ch)
    mesh = plsc.VectorSubcoreMesh(core_axis_name="c", subcore_axis_name="s")
    outs = [jax.ShapeDtypeStruct((_NC, _NP, _D), jnp.float32)]
    scratch = [
        pltpu.VMEM((ch,), jnp.int32),
        pltpu.VMEM((ch,), jnp.int32),
        pltpu.VMEM((ch, _D), jnp.float32),
        pltpu.VMEM((ch, 2 * _D), jnp.float32),
        pltpu.VMEM((ch, _D), jnp.float32),
        pltpu.VMEM_SHARED((_NP, _D), jnp.float32),
    ]
    args = [xl, gb, src3, dst3, zeros]
    if with_count:
        outs.append(jax.ShapeDtypeStruct((_NC, _NP, 16), jnp.float32))
        scratch += [pltpu.VMEM((ch, 16), jnp.float32),
                    pltpu.VMEM_SHARED((_NP, 16), jnp.float32)]
        args.append(zeros16)

    @functools.partial(
        pl.kernel,
        out_type=tuple(outs) if with_count else outs[0],
        mesh=mesh,
        compiler_params=pltpu.CompilerParams(use_tc_tiling_on_sc=False),
        scratch_types=scratch,
    )
    def k(*refs):
        if with_count:
            (xl_hbm, gb_hbm, src_hbm, dst_hbm, zeros_hbm, zeros16_hbm,
             out_hbm, cnt_hbm, src_v, dst_v, xs_v, gb_v, msg_v, acc_sh,
             ones_v, cnt_sh) = refs
        else:
            (xl_hbm, gb_hbm, src_hbm, dst_hbm, zeros_hbm,
             out_hbm, src_v, dst_v, xs_v, gb_v, msg_v, acc_sh) = refs
        c = lax.axis_index("c")
        s = lax.axis_index("s")
        wid = c * _NS + s
        pltpu.sync_copy(zeros_hbm, acc_sh.at[pl.ds(s * _RPT, _RPT)])
        one0 = jnp.maximum(1.0 - lax.iota(jnp.int32, 16).astype(jnp.float32),
                           0.0)
        if with_count:
            pltpu.sync_copy(zeros16_hbm, cnt_sh.at[pl.ds(s * _RPT, _RPT)])

            def fill(e, carry):
                ones_v[e, pl.ds(0, 16)] = one0
                return carry

            lax.fori_loop(0, ch, fill, 0)
        plsc.subcore_barrier()

        def chunk(j, carry):
            pltpu.sync_copy(src_hbm.at[wid, j], src_v)
            pltpu.sync_copy(dst_hbm.at[wid, j], dst_v)
            pltpu.sync_copy(xl_hbm.at[src_v], xs_v)
            pltpu.sync_copy(gb_hbm.at[dst_v], gb_v)

            @plsc.parallel_loop(0, ch, 1, unroll=4)
            def _edge(e):
                for cc in range(8):
                    g = gb_v[e, pl.ds(cc * 16, 16)]
                    b = gb_v[e, pl.ds(_D + cc * 16, 16)]
                    xv = xs_v[e, pl.ds(cc * 16, 16)]
                    msg_v[e, pl.ds(cc * 16, 16)] = jnp.maximum(g * xv + b, 0.0)
            pltpu.sync_copy(msg_v, acc_sh.at[dst_v], add=True)
            if with_count:
                pltpu.sync_copy(ones_v, cnt_sh.at[dst_v], add=True)
            return carry

        lax.fori_loop(0, nchunk, chunk, 0)
        plsc.subcore_barrier()
        pltpu.sync_copy(acc_sh.at[pl.ds(s * _RPT, _RPT)],
                        out_hbm.at[c, pl.ds(s * _RPT, _RPT)])
        if with_count:
            pltpu.sync_copy(cnt_sh.at[pl.ds(s * _RPT, _RPT)],
                            cnt_hbm.at[c, pl.ds(s * _RPT, _RPT)])

    return k(*args)


def _sc_edge_sum(xl, src3, dst3, zeros):
    src3 = src3.reshape(_NW, _NCHUNK, _CH)
    dst3 = dst3.reshape(_NW, _NCHUNK, _CH)
    mesh = plsc.VectorSubcoreMesh(core_axis_name="c", subcore_axis_name="s")

    @functools.partial(
        pl.kernel,
        out_type=jax.ShapeDtypeStruct((_NC, _NP, _D), jnp.float32),
        mesh=mesh,
        compiler_params=pltpu.CompilerParams(use_tc_tiling_on_sc=False),
        scratch_types=[
            pltpu.VMEM((_NCHUNK, _CH), jnp.int32),
            pltpu.VMEM((_NCHUNK, _CH), jnp.int32),
            pltpu.VMEM((_CH, _D), jnp.float32),
            pltpu.VMEM_SHARED((_NP, _D), jnp.float32),
        ],
    )
    def k(xl_hbm, src_hbm, dst_hbm, zeros_hbm, out_hbm,
          src_v, dst_v, xs_v, acc_sh):
        c = lax.axis_index("c")
        s = lax.axis_index("s")
        wid = c * _NS + s
        pltpu.sync_copy(zeros_hbm, acc_sh.at[pl.ds(s * _RPT, _RPT)])
        pltpu.sync_copy(src_hbm.at[wid], src_v)
        pltpu.sync_copy(dst_hbm.at[wid], dst_v)
        plsc.subcore_barrier()

        def chunk(j, carry):
            pltpu.sync_copy(xl_hbm.at[src_v.at[j]], xs_v)
            pltpu.sync_copy(xs_v, acc_sh.at[dst_v.at[j]], add=True)
            return carry

        lax.fori_loop(0, _NCHUNK, chunk, 0)
        plsc.subcore_barrier()
        pltpu.sync_copy(acc_sh.at[pl.ds(s * _RPT, _RPT)],
                        out_hbm.at[c, pl.ds(s * _RPT, _RPT)])

    return k(xl, src3, dst3, zeros)


def _dense(h, Wcat, bfilm, stats, bn_g, bn_b, act):
    with_bn = stats is not None
    args = [h, Wcat, bfilm.reshape(1, 2 * _D)]
    in_specs = [
        pl.BlockSpec((_BLK, _D), lambda i: (i, 0)),
        pl.BlockSpec((_D, 6 * _D), lambda i: (0, 0)),
        pl.BlockSpec((1, 2 * _D), lambda i: (0, 0)),
    ]
    if with_bn:
        args += [stats, bn_g.reshape(1, _D), bn_b.reshape(1, _D)]
        in_specs += [
            pl.BlockSpec((8, _D), lambda i: (0, 0)),
            pl.BlockSpec((1, _D), lambda i: (0, 0)),
            pl.BlockSpec((1, _D), lambda i: (0, 0)),
        ]

    def body(*refs):
        if with_bn:
            h_ref, w_ref, bf_ref, st_ref, g_ref, b_ref, xl_ref, gb_ref, sk_ref = refs
            m = st_ref[0, :] * (1.0 / _N)
            ex2 = st_ref[1, :] * (1.0 / _N)
            rstd = lax.rsqrt(jnp.maximum(ex2 - m * m, 0.0) + _EPS)
            sc = rstd * g_ref[0, :]
            hb = (h_ref[...] - m[None, :]) * sc[None, :] + b_ref[0, :][None, :]
        else:
            h_ref, w_ref, bf_ref, xl_ref, gb_ref, sk_ref = refs
            hb = h_ref[...]
        z = jnp.dot(hb, w_ref[...], preferred_element_type=jnp.float32)
        xl_ref[...] = z[:, :_D]
        beta = z[:, _D:2 * _D] + bf_ref[0, :_D][None, :]
        gamma = z[:, 2 * _D:3 * _D] + bf_ref[0, _D:][None, :]
        gb_ref[...] = jnp.concatenate([gamma, beta], axis=1)
        sk = z[:, 5 * _D:6 * _D] * z[:, 3 * _D:4 * _D] + z[:, 4 * _D:5 * _D]
        if act:
            sk = jnp.maximum(sk, 0.0)
        sk_ref[...] = sk

    return pl.pallas_call(
        body,
        grid=(_NB,),
        in_specs=in_specs,
        out_specs=[
            pl.BlockSpec((_BLK, _D), lambda i: (i, 0)),
            pl.BlockSpec((_BLK, 2 * _D), lambda i: (i, 0)),
            pl.BlockSpec((_BLK, _D), lambda i: (i, 0)),
        ],
        out_shape=[
            jax.ShapeDtypeStruct((_N, _D), jnp.float32),
            jax.ShapeDtypeStruct((_N, 2 * _D), jnp.float32),
            jax.ShapeDtypeStruct((_N, _D), jnp.float32),
        ],
        compiler_params=pltpu.CompilerParams(
            dimension_semantics=("arbitrary",)),
    )(*args)


def _combine(skip, sc_acc, cnt_sc, cnt2):
    first = cnt2 is None
    args = [skip, sc_acc, cnt_sc] if first else [skip, sc_acc, cnt2]
    in_specs = [
        pl.BlockSpec((_BLK, _D), lambda i: (i, 0)),
        pl.BlockSpec((2, _BLK, _D), lambda i: (0, i, 0)),
    ]
    if first:
        in_specs.append(pl.BlockSpec((2, _BLK, 16), lambda i: (0, i, 0)))
    else:
        in_specs.append(pl.BlockSpec((_BLK, _D), lambda i: (i, 0)))

    def body(*refs):
        if first:
            sk_ref, sc_ref, csc_ref, y_ref, cnt_ref, st_ref, acc = refs
        else:
            sk_ref, sc_ref, cin_ref, y_ref, st_ref, acc = refs
        i = pl.program_id(0)
        msum = sc_ref[0, :, :] + sc_ref[1, :, :]
        if first:
            cnt = csc_ref[0, :, 0] + csc_ref[1, :, 0]
            cnt2b = jnp.broadcast_to(cnt[:, None], (_BLK, _D))
            cnt_ref[...] = cnt2b
        else:
            cnt2b = cin_ref[...]
        y = sk_ref[...] + msum / jnp.maximum(cnt2b, 1.0)
        y_ref[...] = y

        @pl.when(i == 0)
        def _():
            acc[...] = jnp.zeros_like(acc)

        acc[0:1, :] += jnp.sum(y, axis=0)[None, :]
        acc[1:2, :] += jnp.sum(y * y, axis=0)[None, :]

        @pl.when(i == _NB - 1)
        def _():
            st_ref[...] = acc[...]

    out_specs = [pl.BlockSpec((_BLK, _D), lambda i: (i, 0))]
    out_shape = [jax.ShapeDtypeStruct((_N, _D), jnp.float32)]
    if first:
        out_specs.append(pl.BlockSpec((_BLK, _D), lambda i: (i, 0)))
        out_shape.append(jax.ShapeDtypeStruct((_N, _D), jnp.float32))
    out_specs.append(pl.BlockSpec((8, _D), lambda i: (0, 0)))
    out_shape.append(jax.ShapeDtypeStruct((8, _D), jnp.float32))

    return pl.pallas_call(
        body,
        grid=(_NB,),
        in_specs=in_specs,
        out_specs=out_specs,
        out_shape=out_shape,
        scratch_shapes=[pltpu.VMEM((8, _D), jnp.float32)],
        compiler_params=pltpu.CompilerParams(
            dimension_semantics=("arbitrary",)),
    )(*args)


def _final(skip, sc_acc, gb, cnt2):
    def body(sk_ref, sc_ref, gb_ref, cnt_ref, o_ref):
        S = sc_ref[0, :, :] + sc_ref[1, :, :]
        cnt = cnt_ref[...]
        gamma = gb_ref[:, :_D]
        beta = gb_ref[:, _D:]
        o_ref[...] = (sk_ref[...] + gamma * S / jnp.maximum(cnt, 1.0)
                      + beta * (cnt > 0.0).astype(jnp.float32))

    return pl.pallas_call(
        body,
        grid=(_NB,),
        in_specs=[
            pl.BlockSpec((_BLK, _D), lambda i: (i, 0)),
            pl.BlockSpec((2, _BLK, _D), lambda i: (0, i, 0)),
            pl.BlockSpec((_BLK, 2 * _D), lambda i: (i, 0)),
            pl.BlockSpec((_BLK, _D), lambda i: (i, 0)),
        ],
        out_specs=pl.BlockSpec((_BLK, _D), lambda i: (i, 0)),
        out_shape=jax.ShapeDtypeStruct((_N, _D), jnp.float32),
        compiler_params=pltpu.CompilerParams(
            dimension_semantics=("arbitrary",)),
    )(skip, sc_acc, gb, cnt2)


def kernel(x, edge_index, W0_lin, W0_film, b0_film, W0_skip, W0_fskip,
           W1_lin, W1_film, b1_film, W1_skip, W1_fskip,
           W2_lin, W2_film, b2_film, W2_skip, W2_fskip,
           bn0_g, bn0_b, bn1_g, bn1_b):
    src_ids = edge_index[0]
    dst_ids = edge_index[1]
    z128 = jnp.zeros((_RPT, _D), jnp.float32)
    z16 = jnp.zeros((_RPT, 16), jnp.float32)
    Wcat0 = jnp.concatenate([W0_lin, W0_film, W0_skip, W0_fskip], axis=1)
    Wcat1 = jnp.concatenate([W1_lin, W1_film, W1_skip, W1_fskip], axis=1)
    Wcat2 = jnp.concatenate([W2_lin, W2_film, W2_skip, W2_fskip], axis=1)

    xl0, gb0, skip0 = _dense(x, Wcat0, b0_film, None, None, None, act=True)
    sc0, cnt_sc = _sc_edge_film(xl0, gb0, src_ids, dst_ids, z128, z16,
                                ch=40, with_count=True)
    y0, cnt2, stats0 = _combine(skip0, sc0, cnt_sc, None)

    xl1, gb1, skip1 = _dense(y0, Wcat1, b1_film, stats0, bn0_g, bn0_b,
                             act=True)
    sc1 = _sc_edge_film(xl1, gb1, src_ids, dst_ids, z128, None,
                        ch=_CH, with_count=False)
    y1, stats1 = _combine(skip1, sc1, None, cnt2)

    xl2, gb2, skip2 = _dense(y1, Wcat2, b2_film, stats1, bn1_g, bn1_b,
                             act=False)
    sc2 = _sc_edge_sum(xl2, src_ids, dst_ids, z128)
    return _final(skip2, sc2, gb2, cnt2)


# speedup vs baseline: 10.4391x; 2.0825x over previous
import functools

import jax
import jax.numpy as jnp
from jax import lax
from jax.experimental import pallas as pl
from jax.experimental.pallas import tpu as pltpu
from jax.experimental.pallas import tpu_sc as plsc

_N = 10000
_E = 320000
_D = 128
_HD = 64
_EPS = 1e-5

_NC = 2
_NS = 16
_NW = _NC * _NS
_EPW = _E // _NW
_CH = 80
_NCHUNK = _EPW // _CH
_NP = 10240
_RPT = _NP // _NS

_NB = 10
_BLK = _N // _NB

_SC_PARAMS = pltpu.CompilerParams(use_tc_tiling_on_sc=False)


def _sc_edge_film_half(xlh, gbh, src3, dst3, zeros64, zeros16, with_count):
    mesh = plsc.VectorSubcoreMesh(core_axis_name="c", subcore_axis_name="s")
    outs = [jax.ShapeDtypeStruct((_NC, _NP, _HD), jnp.float32)]
    scratch = [
        pltpu.VMEM((_NCHUNK, _CH), jnp.int32),
        pltpu.VMEM((_NCHUNK, _CH), jnp.int32),
        pltpu.VMEM((2, _CH, _HD), jnp.float32),
        pltpu.VMEM((2, _CH, 2 * _HD), jnp.float32),
        pltpu.VMEM((2, _CH, _HD), jnp.float32),
        pltpu.VMEM_SHARED((_NP, _HD), jnp.float32),
        pltpu.SemaphoreType.DMA,
        pltpu.SemaphoreType.DMA,
        pltpu.SemaphoreType.DMA,
        pltpu.SemaphoreType.DMA,
        pltpu.SemaphoreType.DMA,
        pltpu.SemaphoreType.DMA,
    ]
    args = [xlh, gbh, src3, dst3, zeros64]
    if with_count:
        outs.append(jax.ShapeDtypeStruct((_NC, _NP, 16), jnp.float32))
        scratch += [
            pltpu.VMEM((_CH, 16), jnp.float32),
            pltpu.VMEM_SHARED((_NP, 16), jnp.float32),
            pltpu.SemaphoreType.DMA,
            pltpu.SemaphoreType.DMA,
        ]
        args.append(zeros16)

    @functools.partial(
        pl.kernel,
        out_type=tuple(outs) if with_count else outs[0],
        mesh=mesh,
        compiler_params=_SC_PARAMS,
        scratch_types=scratch,
    )
    def k(*refs):
        if with_count:
            (xl_hbm, gb_hbm, src_hbm, dst_hbm, zeros_hbm, zeros16_hbm,
             out_hbm, cnt_hbm, src_v, dst_v, xs_v, gb_v, msg_v, acc_sh,
             sx0, sx1, sg0, sg1, ss0, ss1, ones_v, cnt_sh, sc0, sc1) = refs
            sem_cnt = (sc0, sc1)
        else:
            (xl_hbm, gb_hbm, src_hbm, dst_hbm, zeros_hbm,
             out_hbm, src_v, dst_v, xs_v, gb_v, msg_v, acc_sh,
             sx0, sx1, sg0, sg1, ss0, ss1) = refs
        sem_xs = (sx0, sx1)
        sem_gb = (sg0, sg1)
        sem_sc = (ss0, ss1)
        c = lax.axis_index("c")
        s = lax.axis_index("s")
        wid = c * _NS + s
        pltpu.sync_copy(zeros_hbm, acc_sh.at[pl.ds(s * _RPT, _RPT)])
        pltpu.sync_copy(src_hbm.at[wid], src_v)
        pltpu.sync_copy(dst_hbm.at[wid], dst_v)
        if with_count:
            pltpu.sync_copy(zeros16_hbm, cnt_sh.at[pl.ds(s * _RPT, _RPT)])
            one0 = jnp.maximum(
                1.0 - lax.iota(jnp.int32, 16).astype(jnp.float32), 0.0)

            def fill(e, carry):
                ones_v[e, pl.ds(0, 16)] = one0
                return carry

            lax.fori_loop(0, _CH, fill, 0)
        plsc.subcore_barrier()

        def issue_gathers(j, b):
            pltpu.async_copy(xl_hbm.at[src_v.at[j]], xs_v.at[b], sem_xs[b])
            pltpu.async_copy(gb_hbm.at[dst_v.at[j]], gb_v.at[b], sem_gb[b])

        issue_gathers(0, 0)
        issue_gathers(1, 1)
        n_grp = (_NCHUNK + 1) // 2

        def group(jj, carry):
            for b in range(2):
                j = jj * 2 + b

                @pl.when(j < _NCHUNK)
                def _():
                    pltpu.make_async_copy(
                        xl_hbm.at[src_v.at[j]], xs_v.at[b], sem_xs[b]).wait()
                    pltpu.make_async_copy(
                        gb_hbm.at[dst_v.at[j]], gb_v.at[b], sem_gb[b]).wait()

                    @pl.when(j >= 2)
                    def _():
                        pltpu.make_async_copy(
                            msg_v.at[b], acc_sh.at[dst_v.at[j - 2]],
                            sem_sc[b]).wait()
                        if with_count:
                            pltpu.make_async_copy(
                                ones_v, cnt_sh.at[dst_v.at[j - 2]],
                                sem_cnt[b]).wait()

                    @plsc.parallel_loop(0, _CH, 1, unroll=4)
                    def _edge(e):
                        for cc in range(_HD // 16):
                            g = gb_v[b, e, pl.ds(cc * 16, 16)]
                            bb = gb_v[b, e, pl.ds(_HD + cc * 16, 16)]
                            xv = xs_v[b, e, pl.ds(cc * 16, 16)]
                            msg_v[b, e, pl.ds(cc * 16, 16)] = jnp.maximum(
                                g * xv + bb, 0.0)

                    @pl.when(j + 2 < _NCHUNK)
                    def _():
                        issue_gathers(j + 2, b)

                    pltpu.async_copy(msg_v.at[b], acc_sh.at[dst_v.at[j]],
                                     sem_sc[b], add=True)
                    if with_count:
                        pltpu.async_copy(ones_v, cnt_sh.at[dst_v.at[j]],
                                         sem_cnt[b], add=True)
            return carry

        lax.fori_loop(0, n_grp, group, 0)
        pltpu.make_async_copy(
            msg_v.at[0], acc_sh.at[dst_v.at[_NCHUNK - 1]], sem_sc[0]).wait()
        pltpu.make_async_copy(
            msg_v.at[1], acc_sh.at[dst_v.at[_NCHUNK - 2]], sem_sc[1]).wait()
        if with_count:
            pltpu.make_async_copy(
                ones_v, cnt_sh.at[dst_v.at[_NCHUNK - 1]], sem_cnt[0]).wait()
            pltpu.make_async_copy(
                ones_v, cnt_sh.at[dst_v.at[_NCHUNK - 2]], sem_cnt[1]).wait()
        plsc.subcore_barrier()
        pltpu.sync_copy(acc_sh.at[pl.ds(s * _RPT, _RPT)],
                        out_hbm.at[c, pl.ds(s * _RPT, _RPT)])
        if with_count:
            pltpu.sync_copy(cnt_sh.at[pl.ds(s * _RPT, _RPT)],
                            cnt_hbm.at[c, pl.ds(s * _RPT, _RPT)])

    return k(*args)


def _sc_edge_sum(xl, src3, dst3, zeros):
    mesh = plsc.VectorSubcoreMesh(core_axis_name="c", subcore_axis_name="s")

    @functools.partial(
        pl.kernel,
        out_type=jax.ShapeDtypeStruct((_NC, _NP, _D), jnp.float32),
        mesh=mesh,
        compiler_params=_SC_PARAMS,
        scratch_types=[
            pltpu.VMEM((_NCHUNK, _CH), jnp.int32),
            pltpu.VMEM((_NCHUNK, _CH), jnp.int32),
            pltpu.VMEM((2, _CH, _D), jnp.float32),
            pltpu.VMEM_SHARED((_NP, _D), jnp.float32),
            pltpu.SemaphoreType.DMA,
            pltpu.SemaphoreType.DMA,
            pltpu.SemaphoreType.DMA,
            pltpu.SemaphoreType.DMA,
        ],
    )
    def k(xl_hbm, src_hbm, dst_hbm, zeros_hbm, out_hbm,
          src_v, dst_v, xs_v, acc_sh, sg0, sg1, ss0, ss1):
        sem_g = (sg0, sg1)
        sem_s = (ss0, ss1)
        c = lax.axis_index("c")
        s = lax.axis_index("s")
        wid = c * _NS + s
        pltpu.sync_copy(zeros_hbm, acc_sh.at[pl.ds(s * _RPT, _RPT)])
        pltpu.sync_copy(src_hbm.at[wid], src_v)
        pltpu.sync_copy(dst_hbm.at[wid], dst_v)
        plsc.subcore_barrier()

        pltpu.async_copy(xl_hbm.at[src_v.at[0]], xs_v.at[0], sem_g[0])
        n_grp = (_NCHUNK + 1) // 2

        def group(jj, carry):
            for b in range(2):
                j = jj * 2 + b

                @pl.when(j < _NCHUNK)
                def _():
                    pltpu.make_async_copy(
                        xl_hbm.at[src_v.at[j]], xs_v.at[b], sem_g[b]).wait()
                    pltpu.async_copy(xs_v.at[b], acc_sh.at[dst_v.at[j]],
                                     sem_s[b], add=True)

                    @pl.when(j + 1 < _NCHUNK)
                    def _():
                        @pl.when(j >= 1)
                        def _():
                            pltpu.make_async_copy(
                                xs_v.at[1 - b], acc_sh.at[dst_v.at[j - 1]],
                                sem_s[1 - b]).wait()

                        pltpu.async_copy(xl_hbm.at[src_v.at[j + 1]],
                                         xs_v.at[1 - b], sem_g[1 - b])
            return carry

        lax.fori_loop(0, n_grp, group, 0)
        pltpu.make_async_copy(
            xs_v.at[0], acc_sh.at[dst_v.at[_NCHUNK - 1]], sem_s[0]).wait()
        pltpu.make_async_copy(
            xs_v.at[1], acc_sh.at[dst_v.at[_NCHUNK - 2]], sem_s[1]).wait()
        plsc.subcore_barrier()
        pltpu.sync_copy(acc_sh.at[pl.ds(s * _RPT, _RPT)],
                        out_hbm.at[c, pl.ds(s * _RPT, _RPT)])

    return k(xl, src3, dst3, zeros)


def _dense(h, Wcat, bfilm, stats, bn_g, bn_b, act, split):
    with_bn = stats is not None
    args = [h, Wcat, bfilm.reshape(1, 2 * _D)]
    in_specs = [
        pl.BlockSpec((_BLK, _D), lambda i: (i, 0)),
        pl.BlockSpec((_D, 6 * _D), lambda i: (0, 0)),
        pl.BlockSpec((1, 2 * _D), lambda i: (0, 0)),
    ]
    if with_bn:
        args += [stats, bn_g.reshape(1, _D), bn_b.reshape(1, _D)]
        in_specs += [
            pl.BlockSpec((8, _D), lambda i: (0, 0)),
            pl.BlockSpec((1, _D), lambda i: (0, 0)),
            pl.BlockSpec((1, _D), lambda i: (0, 0)),
        ]

    def body(*refs):
        if with_bn:
            h_ref, w_ref, bf_ref, st_ref, g_ref, b_ref = refs[:6]
            o_refs = refs[6:]
            m = st_ref[0, :] * (1.0 / _N)
            ex2 = st_ref[1, :] * (1.0 / _N)
            rstd = lax.rsqrt(jnp.maximum(ex2 - m * m, 0.0) + _EPS)
            sc = rstd * g_ref[0, :]
            hb = (h_ref[...] - m[None, :]) * sc[None, :] + b_ref[0, :][None, :]
        else:
            h_ref, w_ref, bf_ref = refs[:3]
            o_refs = refs[3:]
            hb = h_ref[...]
        z = jnp.dot(hb, w_ref[...], preferred_element_type=jnp.float32)
        beta = z[:, _D:2 * _D] + bf_ref[0, :_D][None, :]
        gamma = z[:, 2 * _D:3 * _D] + bf_ref[0, _D:][None, :]
        sk = z[:, 5 * _D:6 * _D] * z[:, 3 * _D:4 * _D] + z[:, 4 * _D:5 * _D]
        if act:
            sk = jnp.maximum(sk, 0.0)
        if split:
            xla_ref, xlb_ref, gba_ref, gbb_ref, sk_ref = o_refs
            xla_ref[...] = z[:, :_HD]
            xlb_ref[...] = z[:, _HD:_D]
            gba_ref[...] = jnp.concatenate(
                [gamma[:, :_HD], beta[:, :_HD]], axis=1)
            gbb_ref[...] = jnp.concatenate(
                [gamma[:, _HD:], beta[:, _HD:]], axis=1)
        else:
            xl_ref, gb_ref, sk_ref = o_refs
            xl_ref[...] = z[:, :_D]
            gb_ref[...] = jnp.concatenate([gamma, beta], axis=1)
        sk_ref[...] = sk

    if split:
        out_specs = [
            pl.BlockSpec((_BLK, _HD), lambda i: (i, 0)),
            pl.BlockSpec((_BLK, _HD), lambda i: (i, 0)),
            pl.BlockSpec((_BLK, 2 * _HD), lambda i: (i, 0)),
            pl.BlockSpec((_BLK, 2 * _HD), lambda i: (i, 0)),
            pl.BlockSpec((_BLK, _D), lambda i: (i, 0)),
        ]
        out_shape = [
            jax.ShapeDtypeStruct((_N, _HD), jnp.float32),
            jax.ShapeDtypeStruct((_N, _HD), jnp.float32),
            jax.ShapeDtypeStruct((_N, 2 * _HD), jnp.float32),
            jax.ShapeDtypeStruct((_N, 2 * _HD), jnp.float32),
            jax.ShapeDtypeStruct((_N, _D), jnp.float32),
        ]
    else:
        out_specs = [
            pl.BlockSpec((_BLK, _D), lambda i: (i, 0)),
            pl.BlockSpec((_BLK, 2 * _D), lambda i: (i, 0)),
            pl.BlockSpec((_BLK, _D), lambda i: (i, 0)),
        ]
        out_shape = [
            jax.ShapeDtypeStruct((_N, _D), jnp.float32),
            jax.ShapeDtypeStruct((_N, 2 * _D), jnp.float32),
            jax.ShapeDtypeStruct((_N, _D), jnp.float32),
        ]

    return pl.pallas_call(
        body,
        grid=(_NB,),
        in_specs=in_specs,
        out_specs=out_specs,
        out_shape=out_shape,
        compiler_params=pltpu.CompilerParams(
            dimension_semantics=("arbitrary",)),
    )(*args)


def _combine(skip, scA, scB, cnt_sc, cnt2):
    first = cnt2 is None
    args = [skip, scA, scB, cnt_sc] if first else [skip, scA, scB, cnt2]
    in_specs = [
        pl.BlockSpec((_BLK, _D), lambda i: (i, 0)),
        pl.BlockSpec((2, _BLK, _HD), lambda i: (0, i, 0)),
        pl.BlockSpec((2, _BLK, _HD), lambda i: (0, i, 0)),
    ]
    if first:
        in_specs.append(pl.BlockSpec((2, _BLK, 16), lambda i: (0, i, 0)))
    else:
        in_specs.append(pl.BlockSpec((_BLK, _D), lambda i: (i, 0)))

    def body(*refs):
        if first:
            sk_ref, sa_ref, sb_ref, csc_ref, y_ref, cnt_ref, st_ref, acc = refs
        else:
            sk_ref, sa_ref, sb_ref, cin_ref, y_ref, st_ref, acc = refs
        i = pl.program_id(0)
        msum = jnp.concatenate(
            [sa_ref[0, :, :] + sa_ref[1, :, :],
             sb_ref[0, :, :] + sb_ref[1, :, :]], axis=1)
        if first:
            cnt = csc_ref[0, :, 0] + csc_ref[1, :, 0]
            cnt2b = jnp.broadcast_to(cnt[:, None], (_BLK, _D))
            cnt_ref[...] = cnt2b
        else:
            cnt2b = cin_ref[...]
        y = sk_ref[...] + msum / jnp.maximum(cnt2b, 1.0)
        y_ref[...] = y

        @pl.when(i == 0)
        def _():
            acc[...] = jnp.zeros_like(acc)

        acc[0:1, :] += jnp.sum(y, axis=0)[None, :]
        acc[1:2, :] += jnp.sum(y * y, axis=0)[None, :]

        @pl.when(i == _NB - 1)
        def _():
            st_ref[...] = acc[...]

    out_specs = [pl.BlockSpec((_BLK, _D), lambda i: (i, 0))]
    out_shape = [jax.ShapeDtypeStruct((_N, _D), jnp.float32)]
    if first:
        out_specs.append(pl.BlockSpec((_BLK, _D), lambda i: (i, 0)))
        out_shape.append(jax.ShapeDtypeStruct((_N, _D), jnp.float32))
    out_specs.append(pl.BlockSpec((8, _D), lambda i: (0, 0)))
    out_shape.append(jax.ShapeDtypeStruct((8, _D), jnp.float32))

    return pl.pallas_call(
        body,
        grid=(_NB,),
        in_specs=in_specs,
        out_specs=out_specs,
        out_shape=out_shape,
        scratch_shapes=[pltpu.VMEM((8, _D), jnp.float32)],
        compiler_params=pltpu.CompilerParams(
            dimension_semantics=("arbitrary",)),
    )(*args)


def _final(skip, sc_acc, gb, cnt2):
    def body(sk_ref, sc_ref, gb_ref, cnt_ref, o_ref):
        S = sc_ref[0, :, :] + sc_ref[1, :, :]
        cnt = cnt_ref[...]
        gamma = gb_ref[:, :_D]
        beta = gb_ref[:, _D:]
        o_ref[...] = (sk_ref[...] + gamma * S / jnp.maximum(cnt, 1.0)
                      + beta * (cnt > 0.0).astype(jnp.float32))

    return pl.pallas_call(
        body,
        grid=(_NB,),
        in_specs=[
            pl.BlockSpec((_BLK, _D), lambda i: (i, 0)),
            pl.BlockSpec((2, _BLK, _D), lambda i: (0, i, 0)),
            pl.BlockSpec((_BLK, 2 * _D), lambda i: (i, 0)),
            pl.BlockSpec((_BLK, _D), lambda i: (i, 0)),
        ],
        out_specs=pl.BlockSpec((_BLK, _D), lambda i: (i, 0)),
        out_shape=jax.ShapeDtypeStruct((_N, _D), jnp.float32),
        compiler_params=pltpu.CompilerParams(
            dimension_semantics=("arbitrary",)),
    )(skip, sc_acc, gb, cnt2)


def _film_edges(xlA, xlB, gbA, gbB, src3, dst3, z64, z16, with_count):
    if with_count:
        scA, cnt_sc = _sc_edge_film_half(xlA, gbA, src3, dst3, z64, z16, True)
        scB = _sc_edge_film_half(xlB, gbB, src3, dst3, z64, None, False)
        return scA, scB, cnt_sc
    scA = _sc_edge_film_half(xlA, gbA, src3, dst3, z64, None, False)
    scB = _sc_edge_film_half(xlB, gbB, src3, dst3, z64, None, False)
    return scA, scB


def kernel(x, edge_index, W0_lin, W0_film, b0_film, W0_skip, W0_fskip,
           W1_lin, W1_film, b1_film, W1_skip, W1_fskip,
           W2_lin, W2_film, b2_film, W2_skip, W2_fskip,
           bn0_g, bn0_b, bn1_g, bn1_b):
    src3 = edge_index[0].reshape(_NW, _NCHUNK, _CH)
    dst3 = edge_index[1].reshape(_NW, _NCHUNK, _CH)
    z128 = jnp.zeros((_RPT, _D), jnp.float32)
    z64 = jnp.zeros((_RPT, _HD), jnp.float32)
    z16 = jnp.zeros((_RPT, 16), jnp.float32)
    Wcat0 = jnp.concatenate([W0_lin, W0_film, W0_skip, W0_fskip], axis=1)
    Wcat1 = jnp.concatenate([W1_lin, W1_film, W1_skip, W1_fskip], axis=1)
    Wcat2 = jnp.concatenate([W2_lin, W2_film, W2_skip, W2_fskip], axis=1)

    xlA0, xlB0, gbA0, gbB0, skip0 = _dense(
        x, Wcat0, b0_film, None, None, None, act=True, split=True)
    scA0, scB0, cnt_sc = _film_edges(
        xlA0, xlB0, gbA0, gbB0, src3, dst3, z64, z16, with_count=True)
    y0, cnt2, stats0 = _combine(skip0, scA0, scB0, cnt_sc, None)

    xlA1, xlB1, gbA1, gbB1, skip1 = _dense(
        y0, Wcat1, b1_film, stats0, bn0_g, bn0_b, act=True, split=True)
    scA1, scB1 = _film_edges(
        xlA1, xlB1, gbA1, gbB1, src3, dst3, z64, None, with_count=False)
    y1, stats1 = _combine(skip1, scA1, scB1, None, cnt2)

    xl2, gb2, skip2 = _dense(
        y1, Wcat2, b2_film, stats1, bn1_g, bn1_b, act=False, split=False)
    sc2 = _sc_edge_sum(xl2, src3, dst3, z128)
    return _final(skip2, sc2, gb2, cnt2)
